# baseline (speedup 1.0000x reference)
import functools

import jax
import jax.numpy as jnp
from jax import lax
from jax.experimental import pallas as pl
from jax.experimental.pallas import tpu as pltpu
from jax.experimental.pallas import tpu_sc as plsc

VOCAB = 1000000
EMBED_DIM = 64
BATCH = 16384

NUM_CORES = 2
NUM_SUBCORES = 16
NUM_WORKERS = NUM_CORES * NUM_SUBCORES
B_PER_W = BATCH // NUM_WORKERS
CHUNK = 128
N_CHUNKS = B_PER_W // CHUNK
PACK = 2
LANES = 16

_mesh = plsc.VectorSubcoreMesh(core_axis_name="c", subcore_axis_name="s")


@functools.partial(
    pl.kernel,
    mesh=_mesh,
    out_type=jax.ShapeDtypeStruct((BATCH, EMBED_DIM), jnp.float32),
    scratch_types=[
        pltpu.VMEM((B_PER_W,), jnp.int32),
        pltpu.VMEM((N_CHUNKS, CHUNK), jnp.int32),
        pltpu.VMEM((CHUNK, PACK * EMBED_DIM), jnp.float32),
        pltpu.VMEM((CHUNK, PACK * EMBED_DIM), jnp.float32),
        pltpu.VMEM((CHUNK, EMBED_DIM), jnp.float32),
        pltpu.VMEM((CHUNK, EMBED_DIM), jnp.float32),
        pltpu.SemaphoreType.DMA,
        pltpu.SemaphoreType.DMA,
        pltpu.SemaphoreType.DMA,
        pltpu.SemaphoreType.DMA,
    ],
    compiler_params=pltpu.CompilerParams(needs_layout_passes=False),
)
def _gather_kernel(
    table_hbm, idx_hbm, out_hbm, idx_v, pidx_v, pk0, pk1, outb0, outb1,
    gsem0, gsem1, wsem0, wsem1,
):
    wid = lax.axis_index("s") * NUM_CORES + lax.axis_index("c")
    base = wid * B_PER_W
    pks = (pk0, pk1)
    outbs = (outb0, outb1)
    gsems = (gsem0, gsem1)
    wsems = (wsem0, wsem1)

    pltpu.sync_copy(idx_hbm.at[pl.ds(base, B_PER_W)], idx_v)

    for k in range(B_PER_W // LANES):
        v = idx_v[pl.ds(k * LANES, LANES)]
        j, c = divmod(k * LANES, CHUNK)
        pidx_v[j, pl.ds(c, LANES)] = lax.shift_right_logical(v, 1)

    def start_gather(j):
        return pltpu.async_copy(
            table_hbm.at[pidx_v.at[j]], pks[j % 2], gsems[j % 2]
        )

    def extract(j):
        pk = pks[j % 2]
        outb = outbs[j % 2]
        iota = lax.iota(jnp.int32, LANES)
        for g in range(CHUNK // LANES):
            iv = idx_v[pl.ds(j * CHUNK + g * LANES, LANES)]
            off = lax.bitwise_and(iv, 1) * EMBED_DIM
            pv = iota + g * LANES

            def body(c, _, off=off, pv=pv):
                cc = jnp.full((LANES,), c, jnp.int32)
                vals = plsc.load_gather(pk, [pv, off + cc])
                plsc.store_scatter(outb, [pv, cc], vals)
                return 0

            lax.fori_loop(0, EMBED_DIM, body, 0, unroll=4)

    def start_write(j):
        return pltpu.async_copy(
            outbs[j % 2],
            out_hbm.at[pl.ds(base + j * CHUNK, CHUNK)],
            wsems[j % 2],
        )

    writes = [None, None]
    gathers = [None, None]
    gathers[0] = start_gather(0)
    for j in range(N_CHUNKS):
        if j + 1 < N_CHUNKS:
            gathers[(j + 1) % 2] = start_gather(j + 1)
        gathers[j % 2].wait()
        if writes[j % 2] is not None:
            writes[j % 2].wait()
        extract(j)
        writes[j % 2] = start_write(j)
    writes[(N_CHUNKS - 2) % 2].wait()
    writes[(N_CHUNKS - 1) % 2].wait()


def kernel(embed_cache, indices):
    table2 = embed_cache.reshape(VOCAB // PACK, PACK * EMBED_DIM)
    idx = indices.astype(jnp.int32)
    return _gather_kernel(table2, idx)


# speedup vs baseline: 1.7181x; 1.7181x over previous
import functools

import jax
import jax.numpy as jnp
from jax import lax
from jax.experimental import pallas as pl
from jax.experimental.pallas import tpu as pltpu
from jax.experimental.pallas import tpu_sc as plsc

VOCAB = 1000000
EMBED_DIM = 64
BATCH = 16384

NUM_CORES = 2
NUM_SUBCORES = 16
NUM_WORKERS = NUM_CORES * NUM_SUBCORES
B_PER_W = BATCH // NUM_WORKERS
GROUP = 8
LANES = 16
BKT_CAP = B_PER_W + LANES
POS_BITS = 10
ROW_BYTES = EMBED_DIM * 4

_mesh = plsc.VectorSubcoreMesh(core_axis_name="c", subcore_axis_name="s")


@functools.partial(
    pl.kernel,
    mesh=_mesh,
    out_type=jax.ShapeDtypeStruct((BATCH, EMBED_DIM), jnp.float32),
    scratch_types=[
        pltpu.VMEM((B_PER_W,), jnp.int32),
        pltpu.VMEM((GROUP, BKT_CAP), jnp.int32),
        pltpu.VMEM((B_PER_W + LANES, EMBED_DIM), jnp.float32),
        pltpu.SemaphoreType.DMA,
    ],
    compiler_params=pltpu.CompilerParams(needs_layout_passes=False),
)
def _gather_kernel(table_hbm, idx_hbm, out_hbm, idx_v, bkt, rows_v, gsem):
    wid = lax.axis_index("s") * NUM_CORES + lax.axis_index("c")
    base = wid * B_PER_W
    iota = lax.iota(jnp.int32, LANES)

    pltpu.sync_copy(idx_hbm.at[pl.ds(base, B_PER_W)], idx_v)

    dummy = jnp.full((LANES,), B_PER_W, jnp.int32)
    for k in range(GROUP):
        for g in range(BKT_CAP // LANES):
            bkt[k, pl.ds(g * LANES, LANES)] = dummy

    counts = [jnp.int32(0)] * GROUP
    for g in range(B_PER_W // LANES):
        iv = idx_v[pl.ds(g * LANES, LANES)]
        pv = iota + g * LANES
        packed = lax.bitwise_or(lax.shift_left(iv, POS_BITS), pv)
        rv = lax.bitwise_and(iv, GROUP - 1)
        for k in range(GROUP):
            m = rv == k
            plsc.store_compressed(bkt.at[k, pl.ds(counts[k], LANES)], packed, mask=m)
            counts[k] = counts[k] + plsc.all_reduce_population_count(m)[0]

    n_groups = jnp.int32(0)
    for k in range(GROUP):
        gk = lax.shift_right_logical(counts[k] + (LANES - 1), 4)

        def issue(g, _, k=k):
            wv = bkt[k, pl.ds(pl.multiple_of(g * LANES, LANES), LANES)]
            for i in range(LANES):
                w = wv[i]
                p = lax.bitwise_and(w, (1 << POS_BITS) - 1)
                b8 = lax.shift_left(
                    lax.shift_right_logical(w, POS_BITS + 3), 3
                )
                pltpu.async_copy(
                    table_hbm.at[pl.ds(pl.multiple_of(b8, GROUP) + k, 1)],
                    rows_v.at[pl.ds(p, 1)],
                    gsem,
                )
            return 0

        lax.fori_loop(0, gk, issue, 0)
        n_groups = n_groups + gk

    def drain(_, __):
        pltpu.make_async_copy(
            table_hbm.at[pl.ds(0, LANES)], rows_v.at[pl.ds(0, LANES)], gsem
        ).wait()
        return 0

    lax.fori_loop(0, n_groups, drain, 0)
    pltpu.sync_copy(rows_v.at[pl.ds(0, B_PER_W)], out_hbm.at[pl.ds(base, B_PER_W)])


def kernel(embed_cache, indices):
    idx = indices.astype(jnp.int32)
    return _gather_kernel(embed_cache, idx)


# speedup vs baseline: 2.4824x; 1.4448x over previous
import functools

import jax
import jax.numpy as jnp
from jax import lax
from jax.experimental import pallas as pl
from jax.experimental.pallas import tpu as pltpu
from jax.experimental.pallas import tpu_sc as plsc

VOCAB = 1000000
EMBED_DIM = 64
BATCH = 16384

NUM_CORES = 2
NUM_SUBCORES = 16
NUM_WORKERS = NUM_CORES * NUM_SUBCORES
B_PER_W = BATCH // NUM_WORKERS
GROUP = 8
LANES = 16
BKT_CAP = B_PER_W + LANES
POS_BITS = 10
ROW_BYTES = EMBED_DIM * 4

_mesh = plsc.VectorSubcoreMesh(core_axis_name="c", subcore_axis_name="s")


@functools.partial(
    pl.kernel,
    mesh=_mesh,
    out_type=jax.ShapeDtypeStruct((BATCH, EMBED_DIM), jnp.float32),
    scratch_types=[
        pltpu.VMEM((B_PER_W,), jnp.int32),
        pltpu.VMEM((GROUP, BKT_CAP), jnp.int32),
        pltpu.VMEM((B_PER_W + LANES, EMBED_DIM), jnp.float32),
        pltpu.SemaphoreType.DMA,
    ],
    compiler_params=pltpu.CompilerParams(needs_layout_passes=False),
)
def _gather_kernel(table_hbm, idx_hbm, out_hbm, idx_v, bkt, rows_v, gsem):
    wid = lax.axis_index("s") * NUM_CORES + lax.axis_index("c")
    base = wid * B_PER_W
    iota = lax.iota(jnp.int32, LANES)

    pltpu.sync_copy(idx_hbm.at[pl.ds(base, B_PER_W)], idx_v)

    dummy = jnp.full((LANES,), B_PER_W, jnp.int32)
    for k in range(GROUP):
        for g in range(BKT_CAP // LANES):
            bkt[k, pl.ds(g * LANES, LANES)] = dummy

    counts = [jnp.int32(0)] * GROUP
    for g in range(B_PER_W // LANES):
        iv = idx_v[pl.ds(g * LANES, LANES)]
        pv = iota + g * LANES
        packed = lax.bitwise_or(lax.shift_left(iv, POS_BITS), pv)
        rv = lax.bitwise_and(iv, GROUP - 1)
        for k in range(GROUP):
            m = rv == k
            plsc.store_compressed(bkt.at[k, pl.ds(counts[k], LANES)], packed, mask=m)
            counts[k] = counts[k] + plsc.all_reduce_population_count(m)[0]

    n_groups = jnp.int32(0)
    for k in range(GROUP):
        gk = lax.shift_right_logical(counts[k] + (LANES - 1), 4)

        def issue(g, _, k=k):
            wv = bkt[k, pl.ds(pl.multiple_of(g * LANES, LANES), LANES)]
            for i in range(LANES):
                w = wv[i]
                p = lax.bitwise_and(w, (1 << POS_BITS) - 1)
                sv = lax.shift_right_logical(w, POS_BITS + 3)
                pltpu.async_copy(
                    table_hbm.at[sv, k],
                    rows_v.at[p],
                    gsem,
                )
            return 0

        lax.fori_loop(0, gk, issue, 0)
        n_groups = n_groups + gk

    def drain(_, __):
        pltpu.make_async_copy(
            table_hbm.at[pl.ds(0, LANES), 0], rows_v.at[pl.ds(0, LANES)], gsem
        ).wait()
        return 0

    lax.fori_loop(0, n_groups, drain, 0)
    pltpu.sync_copy(rows_v.at[pl.ds(0, B_PER_W)], out_hbm.at[pl.ds(base, B_PER_W)])


def kernel(embed_cache, indices):
    table3 = embed_cache.reshape(VOCAB // GROUP, GROUP, EMBED_DIM)
    idx = indices.astype(jnp.int32)
    return _gather_kernel(table3, idx)
